# baseline (speedup 1.0000x reference)
import functools

import jax
import jax.numpy as jnp
from jax import lax
from jax.experimental import pallas as pl
from jax.experimental.pallas import tpu as pltpu
from jax.experimental.pallas import tpu_sc as plsc

NUM_CORES = 2
NUM_SUBCORES = 16
NUM_WORKERS = NUM_CORES * NUM_SUBCORES

S = 50
D = 64
PAGES_PER_CHUNK = 8
CR = S * PAGES_PER_CHUNK
CPAD = 512
SUBG = 5
GN = CR // SUBG
GROUPS = CR // 16


def _emb_kernel(B0, V2):
    p_per_w = B0 // NUM_WORKERS
    n_chunks = p_per_w // PAGES_PER_CHUNK
    mesh = plsc.VectorSubcoreMesh(core_axis_name="c", subcore_axis_name="s")

    @functools.partial(
        pl.kernel,
        out_type=jax.ShapeDtypeStruct((B0, S, D), jnp.float32),
        mesh=mesh,
        scratch_types=[
            pltpu.VMEM((CPAD,), jnp.int32),
            pltpu.VMEM((CPAD,), jnp.int32),
            pltpu.VMEM((CR, 2 * D), jnp.float32),
            pltpu.VMEM((CR, D), jnp.float32),
            pltpu.SemaphoreType.DMA,
            pltpu.SemaphoreType.DMA,
        ],
        compiler_params=pltpu.CompilerParams(use_tc_tiling_on_sc=True,
                                             needs_layout_passes=False),
    )
    def emb(tokp_hbm, table_hbm, out_hbm, tok_v, pl_v, gath_v, sel_v,
            gsem, osem):
        wid = lax.axis_index("s") * NUM_CORES + lax.axis_index("c")
        page_base = wid * p_per_w

        def stage_and_fire(c):
            pltpu.sync_copy(tokp_hbm.at[wid, c], tok_v)
            for k in range(CR // 16):
                t16 = tok_v[pl.ds(k * 16, 16)]
                pl_v[pl.ds(k * 16, 16)] = lax.shift_right_logical(t16, 1)
            for j in range(SUBG):
                pltpu.async_copy(
                    table_hbm.at[pl_v.at[pl.ds(j * GN, GN)]],
                    gath_v.at[pl.ds(j * GN, GN)],
                    gsem,
                )

        def drain_gather():
            for j in range(SUBG):
                pltpu.make_async_copy(
                    table_hbm.at[pl_v.at[pl.ds(0, GN)]],
                    gath_v.at[pl.ds(j * GN, GN)],
                    gsem,
                ).wait()

        def drain_outs():
            for k in range(PAGES_PER_CHUNK):
                pltpu.make_async_copy(
                    sel_v.at[pl.ds(k * S, S)],
                    out_hbm.at[page_base],
                    osem,
                ).wait()

        def select():
            iota = lax.iota(jnp.int32, 16)

            @pl.loop(0, GROUPS)
            def _grp(g16):
                row16 = iota + g16 * 16
                t16 = tok_v[pl.ds(g16 * 16, 16)]
                col0 = (t16 & 1) * jnp.int32(D)
                for w in range(D):
                    val = plsc.load_gather(gath_v, [row16, col0 + w])
                    plsc.store_scatter(
                        sel_v, [row16, jnp.full((16,), w, jnp.int32)], val)

        def fire_outs(c):
            for k in range(PAGES_PER_CHUNK):
                pltpu.async_copy(
                    sel_v.at[pl.ds(k * S, S)],
                    out_hbm.at[page_base + c * PAGES_PER_CHUNK + k],
                    osem,
                )

        stage_and_fire(0)
        drain_gather()
        select()
        fire_outs(0)
        stage_and_fire(1)

        @pl.loop(1, n_chunks - 1)
        def _steady(c):
            drain_gather()
            drain_outs()
            select()
            fire_outs(c)
            stage_and_fire(c + 1)

        drain_gather()
        drain_outs()
        select()
        fire_outs(n_chunks - 1)
        drain_outs()

    return emb


def kernel(token_ids, ME):
    B0, S_ = token_ids.shape
    V, D_ = ME.shape
    table2 = jnp.reshape(ME, (V // 2, 2 * D_))
    tokp = jnp.pad(
        token_ids.reshape(NUM_WORKERS, -1, CR),
        ((0, 0), (0, 0), (0, CPAD - CR)),
    )
    return _emb_kernel(B0, V // 2)(tokp, table2)


# speedup vs baseline: 2.3990x; 2.3990x over previous
import functools

import jax
import jax.numpy as jnp
from jax import lax
from jax.experimental import pallas as pl
from jax.experimental.pallas import tpu as pltpu
from jax.experimental.pallas import tpu_sc as plsc

NUM_CORES = 2
NUM_SUBCORES = 16
NUM_WORKERS = NUM_CORES * NUM_SUBCORES

NBUF = 8
KAHEAD = 6


def _emb_kernel(B0, S, V, D):
    p_per_w = B0 // NUM_WORKERS
    mesh = plsc.VectorSubcoreMesh(core_axis_name="c", subcore_axis_name="s")

    @functools.partial(
        pl.kernel,
        out_type=jax.ShapeDtypeStruct((B0, S, D), jnp.float32),
        mesh=mesh,
        scratch_types=[
            pltpu.VMEM((p_per_w, S), jnp.int32),
            pltpu.VMEM((NBUF, S, D), jnp.float32),
        ] + [pltpu.SemaphoreType.DMA] * (2 * NBUF),
        compiler_params=pltpu.CompilerParams(use_tc_tiling_on_sc=False),
    )
    def emb(tok_hbm, table_hbm, out_hbm, idx_v, rows_v, *sems):
        gsems, osems = sems[:NBUF], sems[NBUF:]
        wid = lax.axis_index("s") * NUM_CORES + lax.axis_index("c")
        base = wid * p_per_w
        pltpu.sync_copy(tok_hbm.at[pl.ds(base, p_per_w)], idx_v)

        def fire(g, b):
            pltpu.async_copy(table_hbm.at[idx_v.at[g]], rows_v.at[b], gsems[b])

        def drain_gather(b):
            pltpu.make_async_copy(
                table_hbm.at[idx_v.at[0]], rows_v.at[b], gsems[b]).wait()

        def start_out(g, b):
            pltpu.async_copy(rows_v.at[b], out_hbm.at[base + g], osems[b])

        def wait_out(b):
            pltpu.make_async_copy(
                rows_v.at[b], out_hbm.at[base], osems[b]).wait()

        def visit(g, b, bk, do_fire, do_owait):
            if do_fire:
                if do_owait:
                    wait_out(bk)
                fire(g + KAHEAD, bk)
            drain_gather(b)
            start_out(g, b)

        for g in range(KAHEAD):
            fire(g, g % NBUF)
        for g in range(NBUF - KAHEAD):
            visit(g, g % NBUF, (g + KAHEAD) % NBUF, True, False)
        lo, hi = NBUF - KAHEAD, p_per_w - KAHEAD
        assert (hi - lo) % NBUF == 0

        @pl.loop(lo, hi, step=NBUF)
        def _steady(t):
            for i in range(NBUF):
                b = (lo + i) % NBUF
                visit(t + i, b, (b + KAHEAD) % NBUF, True, True)

        for g in range(p_per_w - KAHEAD, p_per_w):
            visit(g, g % NBUF, 0, False, False)
        for b in range(NBUF):
            wait_out(b)

    return emb


def kernel(token_ids, ME):
    B0, S = token_ids.shape
    V, D = ME.shape
    return _emb_kernel(B0, S, V, D)(token_ids, ME)
